# baseline (speedup 1.0000x reference)
import functools
import math

import jax
import jax.numpy as jnp
from jax import lax
from jax.experimental import pallas as pl
from jax.experimental.pallas import tpu as pltpu
from jax.experimental.pallas import tpu_sc as plsc

N = 10000
NP = 10240
EPS = 1e-5

NC = 2
NS = 16
NW = NC * NS
SUB = 128
ROWS_PER_TILE = NP // NS


def _mesh():
    return plsc.VectorSubcoreMesh(core_axis_name="c", subcore_axis_name="s")


def _zero_rows(ref, nrows, d):
    z = jnp.zeros((16,), jnp.float32)

    def body(i, _):
        for k in range(d // 16):
            ref[i, pl.ds(k * 16, 16)] = z
        return 0

    lax.fori_loop(0, nrows, body, 0, unroll=False)


def _zero_acc_slice(rowsb, acc, sid, d):
    base = sid * ROWS_PER_TILE
    for b in range(ROWS_PER_TILE // 128):
        pltpu.sync_copy(rowsb, acc.at[pl.ds(base + b * 128, 128)])


def _make_deg_kernel(n_sub):
    @functools.partial(
        pl.kernel,
        out_type=jax.ShapeDtypeStruct((NC, NP, 16), jnp.float32),
        mesh=_mesh(),
        compiler_params=pltpu.CompilerParams(needs_layout_passes=False),
        scratch_types=[
            pltpu.VMEM((n_sub, SUB), jnp.int32),
            pltpu.VMEM((n_sub, SUB), jnp.float32),
            pltpu.VMEM((SUB, 16), jnp.float32),
            pltpu.VMEM_SHARED((NP, 16), jnp.float32),
        ],
    )
    def deg_kernel(dst_hbm, ew_hbm, out_hbm, dstb, ewb, rowsb, acc):
        cid = lax.axis_index("c")
        sid = lax.axis_index("s")
        wid = cid * NS + sid
        pltpu.sync_copy(dst_hbm.at[wid], dstb)
        pltpu.sync_copy(ew_hbm.at[wid], ewb)
        _zero_rows(rowsb, SUB, 16)
        _zero_acc_slice(rowsb, acc, sid, 16)
        plsc.subcore_barrier()

        def step(j, _):
            def inner(m, _):
                w16 = ewb[j, pl.ds(m * 16, 16)]
                for e in range(16):
                    rowsb[m * 16 + e, :] = jnp.full((16,), w16[e])
                return 0

            lax.fori_loop(0, SUB // 16, inner, 0, unroll=False)
            pltpu.sync_copy(rowsb, acc.at[dstb.at[j]], add=True)
            return 0

        lax.fori_loop(0, n_sub, step, 0, unroll=False)
        plsc.subcore_barrier()
        base = sid * ROWS_PER_TILE
        pltpu.sync_copy(acc.at[pl.ds(base, ROWS_PER_TILE)],
                        out_hbm.at[cid, pl.ds(base, ROWS_PER_TILE)])

    return deg_kernel


def _make_norm_kernel(n_sub):
    @functools.partial(
        pl.kernel,
        out_type=jax.ShapeDtypeStruct((NW, n_sub, SUB), jnp.float32),
        mesh=_mesh(),
        compiler_params=pltpu.CompilerParams(needs_layout_passes=False),
        scratch_types=[
            pltpu.VMEM((NP,), jnp.float32),
            pltpu.VMEM((n_sub, SUB), jnp.int32),
            pltpu.VMEM((n_sub, SUB), jnp.int32),
            pltpu.VMEM((n_sub, SUB), jnp.float32),
            pltpu.VMEM((n_sub, SUB), jnp.float32),
        ],
    )
    def norm_kernel(dinv_hbm, src_hbm, dst_hbm, ew_hbm, out_hbm,
                    dinvb, srcb, dstb, ewb, normb):
        cid = lax.axis_index("c")
        sid = lax.axis_index("s")
        wid = cid * NS + sid
        pltpu.sync_copy(dinv_hbm, dinvb)
        pltpu.sync_copy(src_hbm.at[wid], srcb)
        pltpu.sync_copy(dst_hbm.at[wid], dstb)
        pltpu.sync_copy(ew_hbm.at[wid], ewb)

        def step(j, _):
            def inner(m, _):
                s16 = srcb[j, pl.ds(m * 16, 16)]
                d16 = dstb[j, pl.ds(m * 16, 16)]
                w16 = ewb[j, pl.ds(m * 16, 16)]
                a = plsc.load_gather(dinvb, [s16])
                b = plsc.load_gather(dinvb, [d16])
                normb[j, pl.ds(m * 16, 16)] = a * w16 * b
                return 0

            lax.fori_loop(0, SUB // 16, inner, 0, unroll=True)
            return 0

        lax.fori_loop(0, n_sub, step, 0, unroll=False)
        pltpu.sync_copy(normb, out_hbm.at[wid])

    return norm_kernel


def _make_prop_kernel(n_sub):
    n_pair = n_sub // 2

    @functools.partial(
        pl.kernel,
        out_type=jax.ShapeDtypeStruct((NC, NP, 128), jnp.float32),
        mesh=_mesh(),
        compiler_params=pltpu.CompilerParams(needs_layout_passes=False),
        scratch_types=[
            pltpu.VMEM((n_sub, SUB), jnp.int32),
            pltpu.VMEM((n_sub, SUB), jnp.int32),
            pltpu.VMEM((n_sub, SUB), jnp.float32),
            pltpu.VMEM((SUB, 128), jnp.float32),
            pltpu.VMEM_SHARED((NP, 128), jnp.float32),
        ],
    )
    def prop_kernel(table_hbm, src_hbm, dst_hbm, norm_hbm, out_hbm,
                    srcb, dstb, normb, g0, acc):
        cid = lax.axis_index("c")
        sid = lax.axis_index("s")
        wid = cid * NS + sid
        pltpu.sync_copy(src_hbm.at[wid], srcb)
        pltpu.sync_copy(dst_hbm.at[wid], dstb)
        pltpu.sync_copy(norm_hbm.at[wid], normb)
        _zero_rows(g0, SUB, 128)
        _zero_acc_slice(g0, acc, sid, 128)
        plsc.subcore_barrier()

        def scale_scatter(j, gb):
            def sgroup(m, _):
                nv = normb[j, pl.ds(m * 16, 16)]
                for e in range(16):
                    s = nv[e]
                    r = m * 16 + e
                    for k in range(8):
                        gb[r, pl.ds(k * 16, 16)] = gb[r, pl.ds(k * 16, 16)] * s
                return 0

            lax.fori_loop(0, SUB // 16, sgroup, 0, unroll=False)
            pltpu.sync_copy(gb, acc.at[dstb.at[j]], add=True)

        def step(j, _):
            pltpu.sync_copy(table_hbm.at[srcb.at[j]], g0)
            scale_scatter(j, g0)
            return 0

        lax.fori_loop(0, n_sub, step, 0, unroll=False)
        plsc.subcore_barrier()
        base = sid * ROWS_PER_TILE
        pltpu.sync_copy(acc.at[pl.ds(base, ROWS_PER_TILE)],
                        out_hbm.at[cid, pl.ds(base, ROWS_PER_TILE)])

    return prop_kernel


B0 = 1024


def _mm0_call(deg_parts, x, W0):
    def body(dp_ref, x_ref, w_ref, t_ref, dinv_ref):
        deg = dp_ref[0, :, 0:1] + dp_ref[1, :, 0:1] + 2.0
        dinv_ref[...] = lax.rsqrt(deg)
        t_ref[...] = lax.dot_general(
            x_ref[...], w_ref[...], (((1,), (1,)), ((), ())),
            preferred_element_type=jnp.float32)

    return pl.pallas_call(
        body,
        grid=(NP // B0,),
        in_specs=[
            pl.BlockSpec((NC, B0, 16), lambda i: (0, i, 0)),
            pl.BlockSpec((B0, 128), lambda i: (i, 0)),
            pl.BlockSpec((128, 128), lambda i: (0, 0)),
        ],
        out_specs=[
            pl.BlockSpec((B0, 128), lambda i: (i, 0)),
            pl.BlockSpec((B0, 1), lambda i: (i, 0)),
        ],
        out_shape=[
            jax.ShapeDtypeStruct((NP, 128), jnp.float32),
            jax.ShapeDtypeStruct((NP, 1), jnp.float32),
        ],
    )(deg_parts, x, W0)


def _post_call(parts, table, dinv, resid, b, g, be, Wn):
    has_resid = resid is not None
    bn_scale = 1.0 / math.sqrt(1.0 + EPS)

    def body(*refs):
        if has_resid:
            p_ref, t_ref, dinv_ref, r_ref, b_ref, g_ref, be_ref, w_ref, o_ref = refs
        else:
            p_ref, t_ref, dinv_ref, b_ref, g_ref, be_ref, w_ref, o_ref = refs
        dinv = dinv_ref[...]
        conv = (p_ref[0] + p_ref[1] + (2.0 * dinv * dinv) * t_ref[...]
                + b_ref[0][None, :])
        h = conv * (g_ref[0][None, :] * bn_scale) + be_ref[0][None, :]
        h = jnp.maximum(h, 0.0)
        if has_resid:
            h = h + r_ref[...]
        o_ref[...] = lax.dot_general(
            h, w_ref[...], (((1,), (1,)), ((), ())),
            preferred_element_type=jnp.float32)

    in_specs = [
        pl.BlockSpec((NC, B0, 128), lambda i: (0, i, 0)),
        pl.BlockSpec((B0, 128), lambda i: (i, 0)),
        pl.BlockSpec((B0, 1), lambda i: (i, 0)),
    ]
    args = [parts, table, dinv]
    if has_resid:
        in_specs.append(pl.BlockSpec((B0, 128), lambda i: (i, 0)))
        args.append(resid)
    in_specs += [
        pl.BlockSpec((1, 128), lambda i: (0, 0)),
        pl.BlockSpec((1, 128), lambda i: (0, 0)),
        pl.BlockSpec((1, 128), lambda i: (0, 0)),
        pl.BlockSpec((128, 128), lambda i: (0, 0)),
    ]
    args += [b.reshape(1, 128), g.reshape(1, 128), be.reshape(1, 128), Wn]

    return pl.pallas_call(
        body,
        grid=(NP // B0,),
        in_specs=in_specs,
        out_specs=pl.BlockSpec((B0, 128), lambda i: (i, 0)),
        out_shape=jax.ShapeDtypeStruct((NP, 128), jnp.float32),
    )(*args)


def _head_call(parts, table, dinv, b2, fW1, fb1, fg, fbe, fW2, fb2):
    bn_scale = 1.0 / math.sqrt(1.0 + EPS)

    def body(p_ref, t_ref, dinv_ref, b2_ref, fw1_ref, fb1_ref, fg_ref,
             fbe_ref, fw2_ref, fb2_ref, o_ref):
        dinv = dinv_ref[...]
        h3 = (p_ref[0] + p_ref[1] + (2.0 * dinv * dinv) * t_ref[...]
              + b2_ref[0][None, :])
        valid = (lax.broadcasted_iota(jnp.int32, (NP, 1), 0) < N
                 ).astype(jnp.float32)
        p = jnp.sum(h3 * valid, axis=0, keepdims=True) * (1.0 / N)
        f = lax.dot_general(p, fw1_ref[...], (((1,), (1,)), ((), ())),
                            preferred_element_type=jnp.float32) + fb1_ref[...]
        f = f * (fg_ref[...] * bn_scale) + fbe_ref[...]
        f = jnp.maximum(f, 0.0)
        o_ref[...] = (jnp.sum(f * fw2_ref[...], axis=1, keepdims=True)
                      + fb2_ref[...])

    b2p = jnp.pad(b2, (0, 64)).reshape(1, 128)
    fW1p = jnp.pad(fW1, ((0, 0), (0, 64)))
    return pl.pallas_call(
        body,
        out_shape=jax.ShapeDtypeStruct((1, 1), jnp.float32),
    )(parts, table, dinv, b2p, fW1p, fb1.reshape(1, 64),
      fg.reshape(1, 64), fbe.reshape(1, 64), fW2, fb2.reshape(1, 1))


def kernel(x, edge_index, edge_weight, W0, b0, g0, be0, W1, b1, g1, be1,
           W2, b2, fW1, fb1, fg, fbe, fW2, fb2):
    E = edge_index.shape[1]
    n_sub = -(-E // (NW * SUB))
    n_sub += n_sub % 2
    e_pad = NW * n_sub * SUB - E

    src = edge_index[0]
    dst = edge_index[1]
    if e_pad:
        pad_dst = N + (jnp.arange(e_pad, dtype=jnp.int32) % (NP - N))
        src = jnp.concatenate([src, jnp.zeros((e_pad,), jnp.int32)])
        dst = jnp.concatenate([dst, pad_dst])
        ew = jnp.concatenate([edge_weight, jnp.zeros((e_pad,), jnp.float32)])
    else:
        ew = edge_weight
    src_p = src.reshape(NW, n_sub, SUB)
    dst_p = dst.reshape(NW, n_sub, SUB)
    ew_p = ew.reshape(NW, n_sub, SUB)

    xp = jnp.pad(x, ((0, NP - N), (0, 0)))

    deg_parts = _make_deg_kernel(n_sub)(dst_p, ew_p)
    table0, dinv = _mm0_call(deg_parts, xp, W0)
    norm_p = _make_norm_kernel(n_sub)(dinv.reshape(NP), src_p, dst_p, ew_p)

    prop128 = _make_prop_kernel(n_sub)

    W1p = jnp.pad(W1, ((0, 64), (0, 0)))
    W2p = jnp.pad(W2, ((0, 64), (0, 64)))
    b1p = jnp.pad(b1, (0, 64))
    g1p = jnp.pad(g1, (0, 64))
    be1p = jnp.pad(be1, (0, 64))

    parts0 = prop128(table0, src_p, dst_p, norm_p)
    table1 = _post_call(parts0, table0, dinv, xp, b0, g0, be0, W1p)
    parts1 = prop128(table1, src_p, dst_p, norm_p)
    table2 = _post_call(parts1, table1, dinv, None, b1p, g1p, be1p, W2p)
    parts2 = prop128(table2, src_p, dst_p, norm_p)
    return _head_call(parts2, table2, dinv, b2, fW1, fb1, fg, fbe, fW2, fb2)


# speedup vs baseline: 1.3927x; 1.3927x over previous
import functools
import math

import jax
import jax.numpy as jnp
from jax import lax
from jax.experimental import pallas as pl
from jax.experimental.pallas import tpu as pltpu
from jax.experimental.pallas import tpu_sc as plsc

N = 10000
NP = 10240
EPS = 1e-5

NC = 2
NS = 16
NW = NC * NS
SUB = 128
ROWS_PER_TILE = NP // NS


def _mesh():
    return plsc.VectorSubcoreMesh(core_axis_name="c", subcore_axis_name="s")


def _zero_rows(ref, nrows, d):
    z = jnp.zeros((16,), jnp.float32)

    def body(i, _):
        for k in range(d // 16):
            ref[i, pl.ds(k * 16, 16)] = z
        return 0

    lax.fori_loop(0, nrows, body, 0, unroll=False)


def _zero_acc_slice(rowsb, acc, sid, d):
    base = sid * ROWS_PER_TILE
    for b in range(ROWS_PER_TILE // 128):
        pltpu.sync_copy(rowsb, acc.at[pl.ds(base + b * 128, 128)])


def _make_deg_kernel(n_sub):
    @functools.partial(
        pl.kernel,
        out_type=jax.ShapeDtypeStruct((NC, NP, 16), jnp.float32),
        mesh=_mesh(),
        compiler_params=pltpu.CompilerParams(needs_layout_passes=False),
        scratch_types=[
            pltpu.VMEM((n_sub, SUB), jnp.int32),
            pltpu.VMEM((n_sub, SUB), jnp.float32),
            pltpu.VMEM((SUB, 16), jnp.float32),
            pltpu.VMEM_SHARED((NP, 16), jnp.float32),
        ],
    )
    def deg_kernel(dst_hbm, ew_hbm, out_hbm, dstb, ewb, rowsb, acc):
        cid = lax.axis_index("c")
        sid = lax.axis_index("s")
        wid = cid * NS + sid
        pltpu.sync_copy(dst_hbm.at[wid], dstb)
        pltpu.sync_copy(ew_hbm.at[wid], ewb)
        _zero_rows(rowsb, SUB, 16)
        _zero_acc_slice(rowsb, acc, sid, 16)
        plsc.subcore_barrier()

        def step(j, _):
            def inner(m, _):
                w16 = ewb[j, pl.ds(m * 16, 16)]
                for e in range(16):
                    rowsb[m * 16 + e, :] = jnp.full((16,), w16[e])
                return 0

            lax.fori_loop(0, SUB // 16, inner, 0, unroll=False)
            pltpu.sync_copy(rowsb, acc.at[dstb.at[j]], add=True)
            return 0

        lax.fori_loop(0, n_sub, step, 0, unroll=False)
        plsc.subcore_barrier()
        base = sid * ROWS_PER_TILE
        pltpu.sync_copy(acc.at[pl.ds(base, ROWS_PER_TILE)],
                        out_hbm.at[cid, pl.ds(base, ROWS_PER_TILE)])

    return deg_kernel


def _make_norm_kernel(n_sub):
    @functools.partial(
        pl.kernel,
        out_type=jax.ShapeDtypeStruct((NW, n_sub, SUB), jnp.float32),
        mesh=_mesh(),
        compiler_params=pltpu.CompilerParams(needs_layout_passes=False),
        scratch_types=[
            pltpu.VMEM((NP,), jnp.float32),
            pltpu.VMEM((n_sub, SUB), jnp.int32),
            pltpu.VMEM((n_sub, SUB), jnp.int32),
            pltpu.VMEM((n_sub, SUB), jnp.float32),
            pltpu.VMEM((n_sub, SUB), jnp.float32),
        ],
    )
    def norm_kernel(dinv_hbm, src_hbm, dst_hbm, ew_hbm, out_hbm,
                    dinvb, srcb, dstb, ewb, normb):
        cid = lax.axis_index("c")
        sid = lax.axis_index("s")
        wid = cid * NS + sid
        pltpu.sync_copy(dinv_hbm, dinvb)
        pltpu.sync_copy(src_hbm.at[wid], srcb)
        pltpu.sync_copy(dst_hbm.at[wid], dstb)
        pltpu.sync_copy(ew_hbm.at[wid], ewb)

        def step(j, _):
            def inner(m, _):
                s16 = srcb[j, pl.ds(m * 16, 16)]
                d16 = dstb[j, pl.ds(m * 16, 16)]
                w16 = ewb[j, pl.ds(m * 16, 16)]
                a = plsc.load_gather(dinvb, [s16])
                b = plsc.load_gather(dinvb, [d16])
                normb[j, pl.ds(m * 16, 16)] = a * w16 * b
                return 0

            lax.fori_loop(0, SUB // 16, inner, 0, unroll=True)
            return 0

        lax.fori_loop(0, n_sub, step, 0, unroll=False)
        pltpu.sync_copy(normb, out_hbm.at[wid])

    return norm_kernel


def _make_prop_kernel(n_sub):
    n_pair = n_sub // 2

    @functools.partial(
        pl.kernel,
        out_type=jax.ShapeDtypeStruct((NC, NP, 128), jnp.float32),
        mesh=_mesh(),
        compiler_params=pltpu.CompilerParams(needs_layout_passes=False),
        scratch_types=[
            pltpu.VMEM((n_sub, SUB), jnp.int32),
            pltpu.VMEM((n_sub, SUB), jnp.int32),
            pltpu.VMEM((n_sub, SUB), jnp.float32),
            pltpu.VMEM((SUB, 128), jnp.float32),
            pltpu.VMEM_SHARED((NP, 128), jnp.float32),
        ],
    )
    def prop_kernel(table_hbm, src_hbm, dst_hbm, norm_hbm, out_hbm,
                    srcb, dstb, normb, g0, acc):
        cid = lax.axis_index("c")
        sid = lax.axis_index("s")
        wid = cid * NS + sid
        pltpu.sync_copy(src_hbm.at[wid], srcb)
        pltpu.sync_copy(dst_hbm.at[wid], dstb)
        pltpu.sync_copy(norm_hbm.at[wid], normb)
        _zero_rows(g0, SUB, 128)
        _zero_acc_slice(g0, acc, sid, 128)
        plsc.subcore_barrier()

        def scale_scatter(j, gb):
            def sgroup(m, _):
                nv = normb[j, pl.ds(m * 16, 16)]
                for e in range(16):
                    s = nv[e]
                    r = m * 16 + e
                    for k in range(8):
                        gb[r, pl.ds(k * 16, 16)] = gb[r, pl.ds(k * 16, 16)] * s
                return 0

            lax.fori_loop(0, SUB // 16, sgroup, 0, unroll=False)
            pltpu.sync_copy(gb, acc.at[dstb.at[j]], add=True)

        def step(j, _):
            pltpu.sync_copy(table_hbm.at[srcb.at[j]], g0)
            scale_scatter(j, g0)
            return 0

        lax.fori_loop(0, n_sub, step, 0, unroll=False)
        plsc.subcore_barrier()
        base = sid * ROWS_PER_TILE
        pltpu.sync_copy(acc.at[pl.ds(base, ROWS_PER_TILE)],
                        out_hbm.at[cid, pl.ds(base, ROWS_PER_TILE)])

    return prop_kernel


B0 = 1024


def _mm0_call(deg_parts, x, W0):
    def body(dp_ref, x_ref, w_ref, t_ref, dinv_ref):
        deg = dp_ref[0, :, 0:1] + dp_ref[1, :, 0:1] + 2.0
        dinv_ref[...] = lax.rsqrt(deg)
        t_ref[...] = lax.dot_general(
            x_ref[...], w_ref[...], (((1,), (1,)), ((), ())),
            preferred_element_type=jnp.float32)

    return pl.pallas_call(
        body,
        grid=(NP // B0,),
        in_specs=[
            pl.BlockSpec((NC, B0, 16), lambda i: (0, i, 0)),
            pl.BlockSpec((B0, 128), lambda i: (i, 0)),
            pl.BlockSpec((128, 128), lambda i: (0, 0)),
        ],
        out_specs=[
            pl.BlockSpec((B0, 128), lambda i: (i, 0)),
            pl.BlockSpec((B0, 1), lambda i: (i, 0)),
        ],
        out_shape=[
            jax.ShapeDtypeStruct((NP, 128), jnp.float32),
            jax.ShapeDtypeStruct((NP, 1), jnp.float32),
        ],
    )(deg_parts, x, W0)


def _post_call(parts, table, dinv, resid, b, g, be, Wn):
    has_resid = resid is not None
    bn_scale = 1.0 / math.sqrt(1.0 + EPS)

    def body(*refs):
        if has_resid:
            p_ref, t_ref, dinv_ref, r_ref, b_ref, g_ref, be_ref, w_ref, o_ref = refs
        else:
            p_ref, t_ref, dinv_ref, b_ref, g_ref, be_ref, w_ref, o_ref = refs
        dinv = dinv_ref[...]
        conv = (p_ref[0] + p_ref[1] + (2.0 * dinv * dinv) * t_ref[...]
                + b_ref[0][None, :])
        h = conv * (g_ref[0][None, :] * bn_scale) + be_ref[0][None, :]
        h = jnp.maximum(h, 0.0)
        if has_resid:
            h = h + r_ref[...]
        o_ref[...] = lax.dot_general(
            h, w_ref[...], (((1,), (1,)), ((), ())),
            preferred_element_type=jnp.float32)

    in_specs = [
        pl.BlockSpec((NC, B0, 128), lambda i: (0, i, 0)),
        pl.BlockSpec((B0, 128), lambda i: (i, 0)),
        pl.BlockSpec((B0, 1), lambda i: (i, 0)),
    ]
    args = [parts, table, dinv]
    if has_resid:
        in_specs.append(pl.BlockSpec((B0, 128), lambda i: (i, 0)))
        args.append(resid)
    in_specs += [
        pl.BlockSpec((1, 128), lambda i: (0, 0)),
        pl.BlockSpec((1, 128), lambda i: (0, 0)),
        pl.BlockSpec((1, 128), lambda i: (0, 0)),
        pl.BlockSpec((128, 128), lambda i: (0, 0)),
    ]
    args += [b.reshape(1, 128), g.reshape(1, 128), be.reshape(1, 128), Wn]

    return pl.pallas_call(
        body,
        grid=(NP // B0,),
        in_specs=in_specs,
        out_specs=pl.BlockSpec((B0, 128), lambda i: (i, 0)),
        out_shape=jax.ShapeDtypeStruct((NP, 128), jnp.float32),
    )(*args)


def _head_call(parts, table, dinv, b2, fW1, fb1, fg, fbe, fW2, fb2):
    bn_scale = 1.0 / math.sqrt(1.0 + EPS)

    def body(p_ref, t_ref, dinv_ref, b2_ref, fw1_ref, fb1_ref, fg_ref,
             fbe_ref, fw2_ref, fb2_ref, o_ref):
        dinv = dinv_ref[...]
        h3 = (p_ref[0] + p_ref[1] + (2.0 * dinv * dinv) * t_ref[...]
              + b2_ref[0][None, :])
        valid = (lax.broadcasted_iota(jnp.int32, (NP, 1), 0) < N
                 ).astype(jnp.float32)
        p = jnp.sum(h3 * valid, axis=0, keepdims=True) * (1.0 / N)
        f = lax.dot_general(p, fw1_ref[...], (((1,), (1,)), ((), ())),
                            preferred_element_type=jnp.float32) + fb1_ref[...]
        f = f * (fg_ref[...] * bn_scale) + fbe_ref[...]
        f = jnp.maximum(f, 0.0)
        o_ref[...] = (jnp.sum(f * fw2_ref[...], axis=1, keepdims=True)
                      + fb2_ref[...])

    b2p = jnp.pad(b2, (0, 64)).reshape(1, 128)
    fW1p = jnp.pad(fW1, ((0, 0), (0, 64)))
    return pl.pallas_call(
        body,
        out_shape=jax.ShapeDtypeStruct((1, 1), jnp.float32),
    )(parts, table, dinv, b2p, fW1p, fb1.reshape(1, 64),
      fg.reshape(1, 64), fbe.reshape(1, 64), fW2, fb2.reshape(1, 1))


def kernel(x, edge_index, edge_weight, W0, b0, g0, be0, W1, b1, g1, be1,
           W2, b2, fW1, fb1, fg, fbe, fW2, fb2):
    E = edge_index.shape[1]
    n_sub = -(-E // (NW * SUB))
    e_pad = NW * n_sub * SUB - E

    src = edge_index[0]
    dst = edge_index[1]
    if e_pad:
        pad_dst = N + (jnp.arange(e_pad, dtype=jnp.int32) % (NP - N))
        src = jnp.concatenate([src, jnp.zeros((e_pad,), jnp.int32)])
        dst = jnp.concatenate([dst, pad_dst])
        ew = jnp.concatenate([edge_weight, jnp.zeros((e_pad,), jnp.float32)])
    else:
        ew = edge_weight
    src_p = src.reshape(NW, n_sub, SUB)
    dst_p = dst.reshape(NW, n_sub, SUB)
    ew_p = ew.reshape(NW, n_sub, SUB)

    xp = jnp.pad(x, ((0, NP - N), (0, 0)))

    deg_parts = _make_deg_kernel(n_sub)(dst_p, ew_p)
    table0, dinv = _mm0_call(deg_parts, xp, W0)
    norm_p = _make_norm_kernel(n_sub)(dinv.reshape(NP), src_p, dst_p, ew_p)

    prop128 = _make_prop_kernel(n_sub)

    W1p = jnp.pad(W1, ((0, 64), (0, 0)))
    W2p = jnp.pad(W2, ((0, 64), (0, 64)))
    b1p = jnp.pad(b1, (0, 64))
    g1p = jnp.pad(g1, (0, 64))
    be1p = jnp.pad(be1, (0, 64))

    parts0 = prop128(table0, src_p, dst_p, norm_p)
    table1 = _post_call(parts0, table0, dinv, xp, b0, g0, be0, W1p)
    parts1 = prop128(table1, src_p, dst_p, norm_p)
    table2 = _post_call(parts1, table1, dinv, None, b1p, g1p, be1p, W2p)
    parts2 = prop128(table2, src_p, dst_p, norm_p)
    return _head_call(parts2, table2, dinv, b2, fW1, fb1, fg, fbe, fW2, fb2)
